# baseline (speedup 1.0000x reference)
import functools

import jax
import jax.numpy as jnp
from jax import lax
from jax.experimental import pallas as pl
from jax.experimental.pallas import tpu as pltpu
from jax.experimental.pallas import tpu_sc as plsc

N = 10000
E = 160000
H = 32
N_ATOM = 16
ITERS = 2

NQ = 2504
NP = 4 * NQ

GW = 256
NWIN = E // GW
Q = E // 4
BR = 800

_NC = 2
_NS = 16


@functools.lru_cache(maxsize=1)
def _vector_mesh():
  return plsc.VectorSubcoreMesh(
      core_axis_name="core", subcore_axis_name="subcore",
      num_cores=_NC, num_subcores=_NS)


def _sc_gather(table, idx2d):
  @functools.partial(
      pl.kernel,
      out_type=jax.ShapeDtypeStruct((E, H), jnp.float32),
      mesh=_vector_mesh(),
      compiler_params=pltpu.CompilerParams(use_tc_tiling_on_sc=False),
  )
  def gk(tab_hbm, i_hbm, o_hbm):
    def body(i_vmem, o_vmem):
      pltpu.sync_copy(tab_hbm.at[i_vmem.at[0]], o_vmem)

    pltpu.emit_pipeline(
        body,
        grid=(NWIN,),
        in_specs=[pl.BlockSpec((1, GW), lambda i: (0, i))],
        out_specs=[pl.BlockSpec((GW, H), lambda i: (i, 0))],
        core_axis_name=("core", "subcore"),
        dimension_semantics=(pltpu.PARALLEL,),
    )(i_hbm, o_hbm)

  return gk(table, idx2d)


def _sc_scatter_add(msg, idx2d, zeros_tab):
  @functools.partial(
      pl.kernel,
      out_type=jax.ShapeDtypeStruct((_NC, NP, H), jnp.float32),
      mesh=_vector_mesh(),
      scratch_types=[pltpu.VMEM_SHARED((NP, H), jnp.float32)],
      compiler_params=pltpu.CompilerParams(use_tc_tiling_on_sc=False),
  )
  def sk(m_hbm, i_hbm, z_hbm, o_hbm, acc_shared):
    cid = lax.axis_index("core")
    sid = lax.axis_index("subcore")
    rows = NP // _NS
    sl = pl.ds(sid * rows, rows)
    pltpu.sync_copy(z_hbm, acc_shared.at[sl])
    plsc.subcore_barrier()

    def body(m_vmem, i_vmem):
      pltpu.sync_copy(m_vmem, acc_shared.at[i_vmem.at[0]], add=True)

    pltpu.emit_pipeline(
        body,
        grid=(NWIN,),
        in_specs=[
            pl.BlockSpec((GW, H), lambda i: (i, 0)),
            pl.BlockSpec((1, GW), lambda i: (0, i)),
        ],
        out_specs=[],
        core_axis_name=("core", "subcore"),
        dimension_semantics=(pltpu.PARALLEL,),
    )(m_hbm, i_hbm)

    plsc.subcore_barrier()
    pltpu.sync_copy(acc_shared.at[sl], o_hbm.at[cid].at[sl])

  return sk(msg, idx2d, zeros_tab)


def _msg_body(ea0_ref, ea1_ref, ea2_ref, ea3_ref, xs_ref, w1_ref, b1_ref,
              w2o_ref, rsum_ref, o_ref):
  accs = []
  for q, ea_ref in enumerate((ea0_ref, ea1_ref, ea2_ref, ea3_ref)):
    eh = jnp.maximum(
        jnp.dot(ea_ref[...], w1_ref[...], preferred_element_type=jnp.float32)
        + b1_ref[...], 0.0)
    ehc = jnp.concatenate(
        [eh.astype(jnp.bfloat16),
         jnp.ones((eh.shape[0], 1), jnp.bfloat16)], axis=1)
    we = jnp.dot(ehc, w2o_ref[...],
                 preferred_element_type=jnp.float32).astype(jnp.bfloat16)
    xs_q = xs_ref[:, q * H:(q + 1) * H]
    xsrep = pltpu.repeat(xs_q.astype(jnp.bfloat16), H, axis=1)
    accs.append(jnp.dot(we * xsrep, rsum_ref[...],
                        preferred_element_type=jnp.float32))
  o_ref[...] = jnp.concatenate(accs, axis=1)


def _tc_msg(ea, xs_p, W1, b1r, W2o, rsum):
  def ea_spec(q):
    return pl.BlockSpec((BR, 4), lambda i, q=q: (q * (Q // BR) + i, 0))

  return pl.pallas_call(
      _msg_body,
      grid=(Q // BR,),
      in_specs=[
          ea_spec(0), ea_spec(1), ea_spec(2), ea_spec(3),
          pl.BlockSpec((BR, 128), lambda i: (i, 0)),
          pl.BlockSpec((4, 128), lambda i: (0, 0)),
          pl.BlockSpec((1, 128), lambda i: (0, 0)),
          pl.BlockSpec((129, H * H), lambda i: (0, 0)),
          pl.BlockSpec((H * H, H), lambda i: (0, 0)),
      ],
      out_specs=pl.BlockSpec((BR, 128), lambda i: (i, 0)),
      out_shape=jax.ShapeDtypeStruct((Q, 128), jnp.float32),
  )(ea, ea, ea, ea, xs_p, W1, b1r, W2o, rsum)


def _node_body(agg2_ref, out_ref, h_ref, wr_ref, wir_ref, wiz_ref, win_ref,
               whr_ref, whz_ref, whn_ref, bc_ref, bir_ref, biz_ref, bin_ref,
               bhr_ref, bhz_ref, bhn_ref, o_ref):
  agg = agg2_ref[0] + agg2_ref[1]
  out = out_ref[...]
  h = h_ref[...]
  dot = lambda a, w: jnp.dot(a, w[...], preferred_element_type=jnp.float32)
  m = jnp.maximum(agg + dot(out, wr_ref) + bc_ref[...], 0.0)
  r = jax.nn.sigmoid(dot(m, wir_ref) + bir_ref[...]
                     + dot(h, whr_ref) + bhr_ref[...])
  z = jax.nn.sigmoid(dot(m, wiz_ref) + biz_ref[...]
                     + dot(h, whz_ref) + bhz_ref[...])
  n = jnp.tanh(dot(m, win_ref) + bin_ref[...]
               + r * (dot(h, whn_ref) + bhn_ref[...]))
  o_ref[...] = (1.0 - z) * n + z * h


def _tc_node_update(agg2p, outp, hp, bd_weights, biases):
  return pl.pallas_call(
      _node_body,
      out_shape=jax.ShapeDtypeStruct((NQ, 128), jnp.float32),
  )(agg2p, outp, hp, *bd_weights, *biases)


def _emb_body(x_ref, emb_ref, o_ref):
  outs = []
  for q in range(4):
    a = x_ref[:, q:q + 1]
    oh = (a == lax.broadcasted_iota(jnp.int32, (NQ, N_ATOM), 1)
          ).astype(jnp.float32)
    outs.append(jnp.maximum(
        jnp.dot(oh, emb_ref[...], preferred_element_type=jnp.float32), 0.0))
  o_ref[...] = jnp.concatenate(outs, axis=1)


def _tc_embed(x_packed, emb):
  return pl.pallas_call(
      _emb_body,
      out_shape=jax.ShapeDtypeStruct((NQ, 128), jnp.float32),
  )(x_packed, emb)


def _final_body(h_ref, x_ref, wo1_ref, bo1_ref, wo2_ref, bo2_ref, o_ref):
  for q in range(4):
    hq = h_ref[:, q * H:(q + 1) * H]
    t = jnp.maximum(
        jnp.dot(hq, wo1_ref[...], preferred_element_type=jnp.float32)
        + bo1_ref[...], 0.0)
    o = jnp.dot(t, wo2_ref[...], preferred_element_type=jnp.float32) \
        + bo2_ref[...]
    a = x_ref[:, q:q + 1]
    oh = (a == lax.broadcasted_iota(jnp.int32, (NQ, N_ATOM), 1)
          ).astype(jnp.float32)
    feat = jnp.concatenate([o, oh], axis=1)
    nrm = jnp.sqrt(jnp.sum(feat * feat, axis=1, keepdims=True))
    o_ref[q] = feat / jnp.maximum(nrm, 1e-12)


def _tc_final(hp, x_packed, Wo1, bo1, Wo2, bo2):
  return pl.pallas_call(
      _final_body,
      out_shape=jax.ShapeDtypeStruct((4, NQ, H + N_ATOM), jnp.float32),
  )(hp, x_packed, Wo1, bo1, Wo2, bo2)


def kernel(x, edge_index, edge_attr, emb, W1, b1, W2, b2, W_root, b_conv,
           W_ih, W_hh, b_ih, b_hh, Wo1, bo1, Wo2, bo2):
  f32 = jnp.float32
  src = edge_index[0]
  dst = edge_index[1]
  srcg = 4 * (src % NQ) + src // NQ
  dstg = 4 * (dst % NQ) + dst // NQ
  src2 = srcg.reshape(4, Q).T.reshape(1, E)
  dst2 = dstg.reshape(4, Q).T.reshape(1, E)
  zeros_tab = jnp.zeros((NP // _NS, H), f32)

  xpad = jnp.concatenate([x[:, 0], jnp.zeros((NP - N,), jnp.int32)])
  x_packed = xpad.reshape(4, NQ).T

  b1r = b1.reshape(1, 128)
  W2o = jnp.concatenate([
      W2.reshape(128, H, H).transpose(0, 2, 1).reshape(128, H * H),
      b2.reshape(H, H).T.reshape(1, H * H)], axis=0).astype(jnp.bfloat16)
  rsum = (jnp.arange(H * H, dtype=jnp.int32)[:, None] // H
          == jnp.arange(H, dtype=jnp.int32)[None, :]).astype(jnp.bfloat16)

  eye4 = jnp.eye(4, dtype=f32)
  bd = lambda w: jnp.kron(eye4, w)
  tile4 = lambda v: jnp.tile(v.reshape(1, H), (1, 4))
  W_ihT = W_ih.T
  W_hhT = W_hh.T
  bd_weights = (
      bd(W_root),
      bd(W_ihT[:, :H]), bd(W_ihT[:, H:2 * H]), bd(W_ihT[:, 2 * H:]),
      bd(W_hhT[:, :H]), bd(W_hhT[:, H:2 * H]), bd(W_hhT[:, 2 * H:]),
  )
  biases = (
      tile4(b_conv),
      tile4(b_ih[:H]), tile4(b_ih[H:2 * H]), tile4(b_ih[2 * H:]),
      tile4(b_hh[:H]), tile4(b_hh[H:2 * H]), tile4(b_hh[2 * H:]),
  )
  bo1r = bo1.reshape(1, H)
  bo2r = bo2.reshape(1, H)

  outp = _tc_embed(x_packed, emb)
  hp = outp
  for _ in range(ITERS):
    xs = _sc_gather(jnp.reshape(outp, (NP, H)), src2)
    xs_p = jnp.reshape(xs, (Q, 128))
    msg_p = _tc_msg(edge_attr, xs_p, W1, b1r, W2o, rsum)
    msg = jnp.reshape(msg_p, (E, H))
    agg2 = _sc_scatter_add(msg, dst2, zeros_tab)
    agg2p = jnp.reshape(agg2, (_NC, NQ, 128))
    hp = _tc_node_update(agg2p, outp, hp, bd_weights, biases)
    outp = hp
  fg4 = _tc_final(hp, x_packed, Wo1, bo1r, Wo2, bo2r)
  return jnp.reshape(fg4, (NP, H + N_ATOM))[:N]


# speedup vs baseline: 1.1622x; 1.1622x over previous
import functools

import jax
import jax.numpy as jnp
from jax import lax
from jax.experimental import pallas as pl
from jax.experimental.pallas import tpu as pltpu
from jax.experimental.pallas import tpu_sc as plsc

N = 10000
E = 160000
H = 32
N_ATOM = 16
ITERS = 2

NQ = 2504
NP = 4 * NQ

GW = 256
NWIN = E // GW
Q = E // 4
BR = 800

_NC = 2
_NS = 16


@functools.lru_cache(maxsize=1)
def _vector_mesh():
  return plsc.VectorSubcoreMesh(
      core_axis_name="core", subcore_axis_name="subcore",
      num_cores=_NC, num_subcores=_NS)


def _sc_gather(table, idx2d):
  @functools.partial(
      pl.kernel,
      out_type=jax.ShapeDtypeStruct((E, H), jnp.float32),
      mesh=_vector_mesh(),
      compiler_params=pltpu.CompilerParams(use_tc_tiling_on_sc=False),
  )
  def gk(tab_hbm, i_hbm, o_hbm):
    def body(i_vmem, o_vmem):
      pltpu.sync_copy(tab_hbm.at[i_vmem.at[0]], o_vmem)

    pltpu.emit_pipeline(
        body,
        grid=(NWIN,),
        in_specs=[pl.BlockSpec((1, GW), lambda i: (0, i))],
        out_specs=[pl.BlockSpec((GW, H), lambda i: (i, 0))],
        core_axis_name=("core", "subcore"),
        dimension_semantics=(pltpu.PARALLEL,),
    )(i_hbm, o_hbm)

  return gk(table, idx2d)


def _sc_scatter_add(msg, idx2d, zeros_tab):
  @functools.partial(
      pl.kernel,
      out_type=jax.ShapeDtypeStruct((_NC, NP, H), jnp.float32),
      mesh=_vector_mesh(),
      scratch_types=[pltpu.VMEM_SHARED((NP, H), jnp.float32)],
      compiler_params=pltpu.CompilerParams(use_tc_tiling_on_sc=False),
  )
  def sk(m_hbm, i_hbm, z_hbm, o_hbm, acc_shared):
    cid = lax.axis_index("core")
    sid = lax.axis_index("subcore")
    rows = NP // _NS
    sl = pl.ds(sid * rows, rows)
    pltpu.sync_copy(z_hbm, acc_shared.at[sl])
    plsc.subcore_barrier()

    def body(m_vmem, i_vmem):
      pltpu.sync_copy(m_vmem, acc_shared.at[i_vmem.at[0]], add=True)

    pltpu.emit_pipeline(
        body,
        grid=(NWIN,),
        in_specs=[
            pl.BlockSpec((GW, H), lambda i: (i, 0)),
            pl.BlockSpec((1, GW), lambda i: (0, i)),
        ],
        out_specs=[],
        core_axis_name=("core", "subcore"),
        dimension_semantics=(pltpu.PARALLEL,),
    )(m_hbm, i_hbm)

    plsc.subcore_barrier()
    pltpu.sync_copy(acc_shared.at[sl], o_hbm.at[cid].at[sl])

  return sk(msg, idx2d, zeros_tab)


def _msg_body(ea_ref, xs_ref, w1_ref, b1_ref, w2o_ref, rsum_ref, o_ref):
  accs = []
  for q in range(4):
    eh = jnp.maximum(
        jnp.dot(ea_ref[:, 4 * q:4 * (q + 1)], w1_ref[...],
                preferred_element_type=jnp.float32)
        + b1_ref[...], 0.0)
    ehc = jnp.concatenate(
        [eh.astype(jnp.bfloat16),
         jnp.ones((eh.shape[0], 1), jnp.bfloat16)], axis=1)
    we = jnp.dot(ehc, w2o_ref[...],
                 preferred_element_type=jnp.float32).astype(jnp.bfloat16)
    xs_q = xs_ref[:, q * H:(q + 1) * H]
    xsrep = pltpu.repeat(xs_q.astype(jnp.bfloat16), H, axis=1)
    accs.append(jnp.dot(we * xsrep, rsum_ref[...],
                        preferred_element_type=jnp.float32))
  o_ref[...] = jnp.concatenate(accs, axis=1)


def _tc_msg(ea16, xs_p, W1, b1r, W2o, rsum):
  return pl.pallas_call(
      _msg_body,
      grid=(Q // BR,),
      in_specs=[
          pl.BlockSpec((BR, 16), lambda i: (i, 0)),
          pl.BlockSpec((BR, 128), lambda i: (i, 0)),
          pl.BlockSpec((4, 128), lambda i: (0, 0)),
          pl.BlockSpec((1, 128), lambda i: (0, 0)),
          pl.BlockSpec((129, H * H), lambda i: (0, 0)),
          pl.BlockSpec((H * H, H), lambda i: (0, 0)),
      ],
      out_specs=pl.BlockSpec((BR, 128), lambda i: (i, 0)),
      out_shape=jax.ShapeDtypeStruct((Q, 128), jnp.float32),
  )(ea16, xs_p, W1, b1r, W2o, rsum)


def _node_body(agg2_ref, out_ref, h_ref, wr_ref, wir_ref, wiz_ref, win_ref,
               whr_ref, whz_ref, whn_ref, bc_ref, bir_ref, biz_ref, bin_ref,
               bhr_ref, bhz_ref, bhn_ref, o_ref):
  agg = agg2_ref[0] + agg2_ref[1]
  out = out_ref[...]
  h = h_ref[...]
  dot = lambda a, w: jnp.dot(a, w[...], preferred_element_type=jnp.float32)
  m = jnp.maximum(agg + dot(out, wr_ref) + bc_ref[...], 0.0)
  r = jax.nn.sigmoid(dot(m, wir_ref) + bir_ref[...]
                     + dot(h, whr_ref) + bhr_ref[...])
  z = jax.nn.sigmoid(dot(m, wiz_ref) + biz_ref[...]
                     + dot(h, whz_ref) + bhz_ref[...])
  n = jnp.tanh(dot(m, win_ref) + bin_ref[...]
               + r * (dot(h, whn_ref) + bhn_ref[...]))
  o_ref[...] = (1.0 - z) * n + z * h


def _tc_node_update(agg2p, outp, hp, bd_weights, biases):
  return pl.pallas_call(
      _node_body,
      out_shape=jax.ShapeDtypeStruct((NQ, 128), jnp.float32),
  )(agg2p, outp, hp, *bd_weights, *biases)


def _emb_body(x_ref, emb_ref, o_ref):
  outs = []
  for q in range(4):
    a = x_ref[:, q:q + 1]
    oh = (a == lax.broadcasted_iota(jnp.int32, (NQ, N_ATOM), 1)
          ).astype(jnp.float32)
    outs.append(jnp.maximum(
        jnp.dot(oh, emb_ref[...], preferred_element_type=jnp.float32), 0.0))
  o_ref[...] = jnp.concatenate(outs, axis=1)


def _tc_embed(x_packed, emb):
  return pl.pallas_call(
      _emb_body,
      out_shape=jax.ShapeDtypeStruct((NQ, 128), jnp.float32),
  )(x_packed, emb)


def _final_body(h_ref, x_ref, wo1_ref, bo1_ref, wo2_ref, bo2_ref, o_ref):
  for q in range(4):
    hq = h_ref[:, q * H:(q + 1) * H]
    t = jnp.maximum(
        jnp.dot(hq, wo1_ref[...], preferred_element_type=jnp.float32)
        + bo1_ref[...], 0.0)
    o = jnp.dot(t, wo2_ref[...], preferred_element_type=jnp.float32) \
        + bo2_ref[...]
    a = x_ref[:, q:q + 1]
    oh = (a == lax.broadcasted_iota(jnp.int32, (NQ, N_ATOM), 1)
          ).astype(jnp.float32)
    feat = jnp.concatenate([o, oh], axis=1)
    nrm = jnp.sqrt(jnp.sum(feat * feat, axis=1, keepdims=True))
    o_ref[q] = feat / jnp.maximum(nrm, 1e-12)


def _tc_final(hp, x_packed, Wo1, bo1, Wo2, bo2):
  return pl.pallas_call(
      _final_body,
      out_shape=jax.ShapeDtypeStruct((4, NQ, H + N_ATOM), jnp.float32),
  )(hp, x_packed, Wo1, bo1, Wo2, bo2)


def kernel(x, edge_index, edge_attr, emb, W1, b1, W2, b2, W_root, b_conv,
           W_ih, W_hh, b_ih, b_hh, Wo1, bo1, Wo2, bo2):
  f32 = jnp.float32
  src2 = edge_index[0].reshape(1, E)
  dst2 = edge_index[1].reshape(1, E)
  ea16 = edge_attr.reshape(Q, 16)
  zeros_tab = jnp.zeros((NP // _NS, H), f32)

  xpad = jnp.concatenate([x[:, 0], jnp.zeros((NP - N,), jnp.int32)])
  x_packed = xpad.reshape(NQ, 4)

  b1r = b1.reshape(1, 128)
  W2o = jnp.concatenate([
      W2.reshape(128, H, H).transpose(0, 2, 1).reshape(128, H * H),
      b2.reshape(H, H).T.reshape(1, H * H)], axis=0).astype(jnp.bfloat16)
  rsum = (jnp.arange(H * H, dtype=jnp.int32)[:, None] // H
          == jnp.arange(H, dtype=jnp.int32)[None, :]).astype(jnp.bfloat16)

  eye4 = jnp.eye(4, dtype=f32)
  bd = lambda w: jnp.kron(eye4, w)
  tile4 = lambda v: jnp.tile(v.reshape(1, H), (1, 4))
  W_ihT = W_ih.T
  W_hhT = W_hh.T
  bd_weights = (
      bd(W_root),
      bd(W_ihT[:, :H]), bd(W_ihT[:, H:2 * H]), bd(W_ihT[:, 2 * H:]),
      bd(W_hhT[:, :H]), bd(W_hhT[:, H:2 * H]), bd(W_hhT[:, 2 * H:]),
  )
  biases = (
      tile4(b_conv),
      tile4(b_ih[:H]), tile4(b_ih[H:2 * H]), tile4(b_ih[2 * H:]),
      tile4(b_hh[:H]), tile4(b_hh[H:2 * H]), tile4(b_hh[2 * H:]),
  )
  bo1r = bo1.reshape(1, H)
  bo2r = bo2.reshape(1, H)

  outp = _tc_embed(x_packed, emb)
  hp = outp
  for _ in range(ITERS):
    xs = _sc_gather(jnp.reshape(outp, (NP, H)), src2)
    xs_p = jnp.reshape(xs, (Q, 128))
    msg_p = _tc_msg(ea16, xs_p, W1, b1r, W2o, rsum)
    msg = jnp.reshape(msg_p, (E, H))
    agg2 = _sc_scatter_add(msg, dst2, zeros_tab)
    agg2p = jnp.reshape(agg2, (_NC, NQ, 128))
    hp = _tc_node_update(agg2p, outp, hp, bd_weights, biases)
    outp = hp
  fg4 = _tc_final(hp, x_packed, Wo1, bo1r, Wo2, bo2r)
  return jnp.reshape(jnp.transpose(fg4, (1, 0, 2)), (NP, H + N_ATOM))[:N]


# speedup vs baseline: 1.1936x; 1.0270x over previous
import functools

import jax
import jax.numpy as jnp
from jax import lax
from jax.experimental import pallas as pl
from jax.experimental.pallas import tpu as pltpu
from jax.experimental.pallas import tpu_sc as plsc

N = 10000
E = 160000
H = 32
N_ATOM = 16
ITERS = 2

NQ = 2504
NP = 4 * NQ

GW = 256
NWIN = E // GW
Q = E // 4
BR = 1600

_NC = 2
_NS = 16


@functools.lru_cache(maxsize=1)
def _vector_mesh():
  return plsc.VectorSubcoreMesh(
      core_axis_name="core", subcore_axis_name="subcore",
      num_cores=_NC, num_subcores=_NS)


def _sc_gather(table, idx2d):
  @functools.partial(
      pl.kernel,
      out_type=jax.ShapeDtypeStruct((E, H), jnp.float32),
      mesh=_vector_mesh(),
      compiler_params=pltpu.CompilerParams(use_tc_tiling_on_sc=False),
  )
  def gk(tab_hbm, i_hbm, o_hbm):
    def body(i_vmem, o_vmem):
      pltpu.sync_copy(tab_hbm.at[i_vmem.at[0]], o_vmem)

    pltpu.emit_pipeline(
        body,
        grid=(NWIN,),
        in_specs=[pl.BlockSpec((1, GW), lambda i: (0, i))],
        out_specs=[pl.BlockSpec((GW, H), lambda i: (i, 0))],
        core_axis_name=("core", "subcore"),
        dimension_semantics=(pltpu.PARALLEL,),
    )(i_hbm, o_hbm)

  return gk(table, idx2d)


def _sc_scatter_add(msg, idx2d, zeros_tab):
  @functools.partial(
      pl.kernel,
      out_type=jax.ShapeDtypeStruct((_NC, NP, H), jnp.float32),
      mesh=_vector_mesh(),
      scratch_types=[pltpu.VMEM_SHARED((NP, H), jnp.float32)],
      compiler_params=pltpu.CompilerParams(use_tc_tiling_on_sc=False),
  )
  def sk(m_hbm, i_hbm, z_hbm, o_hbm, acc_shared):
    cid = lax.axis_index("core")
    sid = lax.axis_index("subcore")
    rows = NP // _NS
    sl = pl.ds(sid * rows, rows)
    pltpu.sync_copy(z_hbm, acc_shared.at[sl])
    plsc.subcore_barrier()

    def body(m_vmem, i_vmem):
      pltpu.sync_copy(m_vmem, acc_shared.at[i_vmem.at[0]], add=True)

    pltpu.emit_pipeline(
        body,
        grid=(NWIN,),
        in_specs=[
            pl.BlockSpec((GW, H), lambda i: (i, 0)),
            pl.BlockSpec((1, GW), lambda i: (0, i)),
        ],
        out_specs=[],
        core_axis_name=("core", "subcore"),
        dimension_semantics=(pltpu.PARALLEL,),
    )(m_hbm, i_hbm)

    plsc.subcore_barrier()
    pltpu.sync_copy(acc_shared.at[sl], o_hbm.at[cid].at[sl])

  return sk(msg, idx2d, zeros_tab)


def _msg_body(ea_ref, xs_ref, w1_ref, b1_ref, w2o_ref, rsum_ref, o_ref):
  accs = []
  for q in range(4):
    eh = jnp.maximum(
        jnp.dot(ea_ref[:, 4 * q:4 * (q + 1)], w1_ref[...],
                preferred_element_type=jnp.float32)
        + b1_ref[...], 0.0)
    ehc = jnp.concatenate(
        [eh.astype(jnp.bfloat16),
         jnp.ones((eh.shape[0], 1), jnp.bfloat16)], axis=1)
    we = jnp.dot(ehc, w2o_ref[...],
                 preferred_element_type=jnp.float32).astype(jnp.bfloat16)
    xs_q = xs_ref[:, q * H:(q + 1) * H]
    xsrep = pltpu.repeat(xs_q.astype(jnp.bfloat16), H, axis=1)
    accs.append(jnp.dot(we * xsrep, rsum_ref[...],
                        preferred_element_type=jnp.float32))
  o_ref[...] = jnp.concatenate(accs, axis=1)


def _tc_msg(ea16, xs_p, W1, b1r, W2o, rsum):
  return pl.pallas_call(
      _msg_body,
      grid=(Q // BR,),
      in_specs=[
          pl.BlockSpec((BR, 16), lambda i: (i, 0)),
          pl.BlockSpec((BR, 128), lambda i: (i, 0)),
          pl.BlockSpec((4, 128), lambda i: (0, 0)),
          pl.BlockSpec((1, 128), lambda i: (0, 0)),
          pl.BlockSpec((129, H * H), lambda i: (0, 0)),
          pl.BlockSpec((H * H, H), lambda i: (0, 0)),
      ],
      out_specs=pl.BlockSpec((BR, 128), lambda i: (i, 0)),
      out_shape=jax.ShapeDtypeStruct((Q, 128), jnp.float32),
  )(ea16, xs_p, W1, b1r, W2o, rsum)


def _node_body(agg2_ref, out_ref, h_ref, wr_ref, wir_ref, wiz_ref, win_ref,
               whr_ref, whz_ref, whn_ref, bc_ref, bir_ref, biz_ref, bin_ref,
               bhr_ref, bhz_ref, bhn_ref, o_ref):
  agg = agg2_ref[0] + agg2_ref[1]
  out = out_ref[...]
  h = h_ref[...]
  dot = lambda a, w: jnp.dot(a, w[...], preferred_element_type=jnp.float32)
  m = jnp.maximum(agg + dot(out, wr_ref) + bc_ref[...], 0.0)
  r = jax.nn.sigmoid(dot(m, wir_ref) + bir_ref[...]
                     + dot(h, whr_ref) + bhr_ref[...])
  z = jax.nn.sigmoid(dot(m, wiz_ref) + biz_ref[...]
                     + dot(h, whz_ref) + bhz_ref[...])
  n = jnp.tanh(dot(m, win_ref) + bin_ref[...]
               + r * (dot(h, whn_ref) + bhn_ref[...]))
  o_ref[...] = (1.0 - z) * n + z * h


def _tc_node_update(agg2p, outp, hp, bd_weights, biases):
  return pl.pallas_call(
      _node_body,
      out_shape=jax.ShapeDtypeStruct((NQ, 128), jnp.float32),
  )(agg2p, outp, hp, *bd_weights, *biases)


def _emb_body(x_ref, emb_ref, o_ref):
  outs = []
  for q in range(4):
    a = x_ref[:, q:q + 1]
    oh = (a == lax.broadcasted_iota(jnp.int32, (NQ, N_ATOM), 1)
          ).astype(jnp.float32)
    outs.append(jnp.maximum(
        jnp.dot(oh, emb_ref[...], preferred_element_type=jnp.float32), 0.0))
  o_ref[...] = jnp.concatenate(outs, axis=1)


def _tc_embed(x_packed, emb):
  return pl.pallas_call(
      _emb_body,
      out_shape=jax.ShapeDtypeStruct((NQ, 128), jnp.float32),
  )(x_packed, emb)


def _final_body(h_ref, x_ref, wo1_ref, bo1_ref, wo2_ref, bo2_ref, o_ref):
  for q in range(4):
    hq = h_ref[:, q * H:(q + 1) * H]
    t = jnp.maximum(
        jnp.dot(hq, wo1_ref[...], preferred_element_type=jnp.float32)
        + bo1_ref[...], 0.0)
    o = jnp.dot(t, wo2_ref[...], preferred_element_type=jnp.float32) \
        + bo2_ref[...]
    a = x_ref[:, q:q + 1]
    oh = (a == lax.broadcasted_iota(jnp.int32, (NQ, N_ATOM), 1)
          ).astype(jnp.float32)
    feat = jnp.concatenate([o, oh], axis=1)
    nrm = jnp.sqrt(jnp.sum(feat * feat, axis=1, keepdims=True))
    o_ref[q] = feat / jnp.maximum(nrm, 1e-12)


def _tc_final(hp, x_packed, Wo1, bo1, Wo2, bo2):
  return pl.pallas_call(
      _final_body,
      out_shape=jax.ShapeDtypeStruct((4, NQ, H + N_ATOM), jnp.float32),
  )(hp, x_packed, Wo1, bo1, Wo2, bo2)


def kernel(x, edge_index, edge_attr, emb, W1, b1, W2, b2, W_root, b_conv,
           W_ih, W_hh, b_ih, b_hh, Wo1, bo1, Wo2, bo2):
  f32 = jnp.float32
  src2 = edge_index[0].reshape(1, E)
  dst2 = edge_index[1].reshape(1, E)
  ea16 = edge_attr.reshape(Q, 16)
  zeros_tab = jnp.zeros((NP // _NS, H), f32)

  xpad = jnp.concatenate([x[:, 0], jnp.zeros((NP - N,), jnp.int32)])
  x_packed = xpad.reshape(NQ, 4)

  b1r = b1.reshape(1, 128)
  W2o = jnp.concatenate([
      W2.reshape(128, H, H).transpose(0, 2, 1).reshape(128, H * H),
      b2.reshape(H, H).T.reshape(1, H * H)], axis=0).astype(jnp.bfloat16)
  rsum = (jnp.arange(H * H, dtype=jnp.int32)[:, None] // H
          == jnp.arange(H, dtype=jnp.int32)[None, :]).astype(jnp.bfloat16)

  eye4 = jnp.eye(4, dtype=f32)
  bd = lambda w: jnp.kron(eye4, w)
  tile4 = lambda v: jnp.tile(v.reshape(1, H), (1, 4))
  W_ihT = W_ih.T
  W_hhT = W_hh.T
  bd_weights = (
      bd(W_root),
      bd(W_ihT[:, :H]), bd(W_ihT[:, H:2 * H]), bd(W_ihT[:, 2 * H:]),
      bd(W_hhT[:, :H]), bd(W_hhT[:, H:2 * H]), bd(W_hhT[:, 2 * H:]),
  )
  biases = (
      tile4(b_conv),
      tile4(b_ih[:H]), tile4(b_ih[H:2 * H]), tile4(b_ih[2 * H:]),
      tile4(b_hh[:H]), tile4(b_hh[H:2 * H]), tile4(b_hh[2 * H:]),
  )
  bo1r = bo1.reshape(1, H)
  bo2r = bo2.reshape(1, H)

  outp = _tc_embed(x_packed, emb)
  hp = outp
  for _ in range(ITERS):
    xs = _sc_gather(jnp.reshape(outp, (NP, H)), src2)
    xs_p = jnp.reshape(xs, (Q, 128))
    msg_p = _tc_msg(ea16, xs_p, W1, b1r, W2o, rsum)
    msg = jnp.reshape(msg_p, (E, H))
    agg2 = _sc_scatter_add(msg, dst2, zeros_tab)
    agg2p = jnp.reshape(agg2, (_NC, NQ, 128))
    hp = _tc_node_update(agg2p, outp, hp, bd_weights, biases)
    outp = hp
  fg4 = _tc_final(hp, x_packed, Wo1, bo1r, Wo2, bo2r)
  return jnp.reshape(jnp.transpose(fg4, (1, 0, 2)), (NP, H + N_ATOM))[:N]
